# baseline (speedup 1.0000x reference)
import jax
import jax.numpy as jnp
from jax.experimental import pallas as pl
from jax.experimental.pallas import tpu as pltpu

_SLOPE = 0.3
_EPS = 1e-5


def _leaky(v):
    return jnp.maximum(v, _SLOPE * v)


def _block_kernel(x_ref, w1_ref, t1_ref, w2_ref, t2_ref, o_ref, *scratches):
    PB, D, H, R, NC = x_ref.shape
    C = NC // 2
    KW = 4 * C
    M = D * H * R
    bf16 = jnp.bfloat16

    @pl.when(pl.program_id(1) == 0)
    def _zero_halos():
        zd = jnp.zeros((1, H + 2, R, KW), bf16)
        zh = jnp.zeros((D, 1, R, KW), bf16)
        zc = jnp.zeros((D, H, 2, C), bf16)
        for ref in scratches:
            ref[0:1, :, :, :] = zd
            ref[D + 1:D + 2, :, :, :] = zd
            ref[1:1 + D, 0:1, :, :] = zh
            ref[1:1 + D, H + 1:H + 2, :, :] = zh
            ref[1:1 + D, 1:1 + H, 0:2, 0:C] = zc
            ref[1:1 + D, 1:1 + H, R - 2:R, KW - C:KW] = zc

    def conv(src_ref, w_ref):
        acc = jnp.zeros((M, NC), jnp.float32)
        for t in range(9):
            kd, kh = t // 3, t % 3
            lhs = src_ref[kd:kd + D, kh:kh + H, :, :].reshape(M, KW)
            acc = acc + jnp.dot(lhs, w_ref[t],
                                preferred_element_type=jnp.float32)
        return acc

    def scatter_windows(vb, dst_ref):
        dst_ref[1:1 + D, 1:1 + H, :, C:3 * C] = vb
        dst_ref[1:1 + D, 1:1 + H, 0:R - 2, 3 * C:KW] = vb[:, :, 2:R, 0:C]
        dst_ref[1:1 + D, 1:1 + H, 2:R, 0:C] = vb[:, :, 0:R - 2, C:NC]

    def conv1_scatter(p, xw_ref, yw_ref):
        scatter_windows(x_ref[p].astype(bf16), xw_ref)
        y = _leaky(conv(xw_ref, w1_ref) + t1_ref[...])
        scatter_windows(y.astype(bf16).reshape(D, H, R, NC), yw_ref)

    def conv2_out(p, yw_ref):
        z = conv(yw_ref, w2_ref) + t2_ref[...] + x_ref[p].reshape(M, NC)
        o_ref[p] = _leaky(z).reshape(D, H, R, NC)

    for p in range(PB):
        conv1_scatter(p, scratches[2 * p], scratches[2 * p + 1])
    for p in range(PB):
        conv2_out(p, scratches[2 * p + 1])


def _build_call(P, D, H, R, NC, PB):
    C = NC // 2
    KW = 4 * C
    steps = P // PB
    half = max(steps // 2, 1)
    ncore = steps // half
    vol = pl.BlockSpec((PB, D, H, R, NC), lambda i, j: (i * half + j, 0, 0, 0, 0))
    wspec = pl.BlockSpec((9, KW, NC), lambda i, j: (0, 0, 0))
    tspec = pl.BlockSpec((1, NC), lambda i, j: (0, 0))
    scratch = pltpu.VMEM((D + 2, H + 2, R, KW), jnp.bfloat16)
    return pl.pallas_call(
        _block_kernel,
        out_shape=jax.ShapeDtypeStruct((P, D, H, R, NC), jnp.float32),
        grid=(ncore, half),
        in_specs=[vol, wspec, tspec, wspec, tspec],
        out_specs=vol,
        scratch_shapes=[scratch] * (2 * PB),
        compiler_params=pltpu.CompilerParams(
            dimension_semantics=("parallel", "arbitrary"),
            vmem_limit_bytes=52 * 1024 * 1024,
        ),
    )


def _fold_band(w, conv_b, gamma, beta, mean, var, C):
    s = gamma * jax.lax.rsqrt(var + _EPS)
    t = conv_b * s + beta - mean * s
    wt = jnp.transpose(w * s[:, None, None, None, None],
                       (2, 3, 4, 1, 0))
    band = jnp.zeros((3, 3, 4, C, 2, C), jnp.float32)
    for wq in range(2):
        for kw in range(3):
            band = band.at[:, :, wq + kw, :, wq, :].set(wt[:, :, kw])
    band = band.reshape(9, 4 * C, 2 * C).astype(jnp.bfloat16)
    tcol = jnp.concatenate([t, t]).reshape(1, 2 * C).astype(jnp.float32)
    return band, tcol


def kernel(x, w1, b1, gamma1, beta1, mean1, var1,
           w2, b2, gamma2, beta2, mean2, var2):
    N, C, D, H, W = x.shape
    NQ = W // 2
    P = N // 2
    xr = x.reshape(P, 2, C, D, H, NQ, 2)
    xr = jnp.transpose(xr, (0, 3, 4, 5, 1, 6, 2))
    xr = xr.reshape(P, D, H, 2 * NQ, 2 * C).astype(jnp.float32)
    band1, t1c = _fold_band(w1, b1, gamma1, beta1, mean1, var1, C)
    band2, t2c = _fold_band(w2, b2, gamma2, beta2, mean2, var2, C)
    PB = 4 if P % 4 == 0 else (2 if P % 2 == 0 else 1)
    out = _build_call(P, D, H, 2 * NQ, 2 * C, PB)(xr, band1, t1c, band2, t2c)
    out = out.reshape(P, D, H, NQ, 2, 2, C)
    out = jnp.transpose(out, (0, 4, 6, 1, 2, 3, 5))
    return out.reshape(N, C, D, H, W)


# speedup vs baseline: 1.1977x; 1.1977x over previous
import jax
import jax.numpy as jnp
from jax.experimental import pallas as pl
from jax.experimental.pallas import tpu as pltpu

_SLOPE = 0.3
_EPS = 1e-5


def _leaky(v):
    return jnp.maximum(v, _SLOPE * v)


def _block_kernel(x_ref, w1_ref, t1_ref, w2_ref, t2_ref, o_ref, *scratches):
    NB, D, H, W, C = x_ref.shape
    NQ = W // 2
    KW = 4 * C
    NC = 2 * C
    R = 2 * NQ
    M = D * H * R
    bf16 = jnp.bfloat16

    @pl.when(pl.program_id(1) == 0)
    def _zero_halos():
        zd = jnp.zeros((1, H + 2, R, KW), bf16)
        zh = jnp.zeros((D, 1, R, KW), bf16)
        zc = jnp.zeros((D, H, 2, C), bf16)
        for ref in scratches:
            ref[0:1, :, :, :] = zd
            ref[D + 1:D + 2, :, :, :] = zd
            ref[1:1 + D, 0:1, :, :] = zh
            ref[1:1 + D, H + 1:H + 2, :, :] = zh
            ref[1:1 + D, 1:1 + H, 0:2, 0:C] = zc
            ref[1:1 + D, 1:1 + H, R - 2:R, KW - C:KW] = zc

    def conv(src_ref, w_ref):
        acc = jnp.zeros((M, NC), jnp.float32)
        for t in range(9):
            kd, kh = t // 3, t % 3
            lhs = src_ref[kd:kd + D, kh:kh + H, :, :].reshape(M, KW)
            acc = acc + jnp.dot(lhs, w_ref[t],
                                preferred_element_type=jnp.float32)
        return acc

    def build_windows(i, xw_ref):
        xv = x_ref[i:i + 2].astype(bf16)
        for q in range(1, NQ - 1):
            win = xv[:, :, :, 2 * q - 1:2 * q + 3, :].reshape(2, D, H, KW)
            xw_ref[1:1 + D, 1:1 + H, 2 * q:2 * q + 2, :] = (
                jnp.transpose(win, (1, 2, 0, 3)))
        w0 = xv[:, :, :, 0:3, :].reshape(2, D, H, 3 * C)
        xw_ref[1:1 + D, 1:1 + H, 0:2, C:KW] = jnp.transpose(w0, (1, 2, 0, 3))
        wl = xv[:, :, :, W - 3:W, :].reshape(2, D, H, 3 * C)
        xw_ref[1:1 + D, 1:1 + H, R - 2:R, 0:3 * C] = jnp.transpose(wl, (1, 2, 0, 3))

    def conv1_scatter(xw_ref, yw_ref):
        y = _leaky(conv(xw_ref, w1_ref) + t1_ref[...])
        yb = y.astype(bf16).reshape(D, H, R, NC)
        yw_ref[1:1 + D, 1:1 + H, :, C:3 * C] = yb
        yc = yb.reshape(D, H, R, 2, C)
        yw_ref[1:1 + D, 1:1 + H, 0:R - 2, 3 * C:KW] = yc[:, :, 2:R, 0, :]
        yw_ref[1:1 + D, 1:1 + H, 2:R, 0:C] = yc[:, :, 0:R - 2, 1, :]

    def conv2_out(i, yw_ref):
        z = conv(yw_ref, w2_ref) + t2_ref[...]
        zs = z.reshape(D, H, NQ, 2, 2, C)
        for n2 in range(2):
            zn = zs[:, :, :, n2, :, :].reshape(D, H, W, C)
            o_ref[i + n2] = _leaky(zn + x_ref[i + n2])

    npairs = NB // 2
    xws = scratches[0::2]
    yws = scratches[1::2]
    for p in range(npairs):
        build_windows(2 * p, xws[p])
    for p in range(npairs):
        conv1_scatter(xws[p], yws[p])
    for p in range(npairs):
        conv2_out(2 * p, yws[p])


def _build_call(N, D, H, W, C, NB):
    NQ = W // 2
    KW, NC = 4 * C, 2 * C
    steps = N // NB
    half = max(steps // 2, 1)
    ncore = steps // half
    vol = pl.BlockSpec((NB, D, H, W, C), lambda i, j: (i * half + j, 0, 0, 0, 0))
    wspec = pl.BlockSpec((9, KW, NC), lambda i, j: (0, 0, 0))
    tspec = pl.BlockSpec((1, NC), lambda i, j: (0, 0))
    scratch = pltpu.VMEM((D + 2, H + 2, 2 * NQ, KW), jnp.bfloat16)
    return pl.pallas_call(
        _block_kernel,
        out_shape=jax.ShapeDtypeStruct((N, D, H, W, C), jnp.float32),
        grid=(ncore, half),
        in_specs=[vol, wspec, tspec, wspec, tspec],
        out_specs=vol,
        scratch_shapes=[scratch] * NB,
        compiler_params=pltpu.CompilerParams(
            dimension_semantics=("parallel", "arbitrary"),
            vmem_limit_bytes=52 * 1024 * 1024,
        ),
    )


def _fold_band(w, conv_b, gamma, beta, mean, var, C):
    s = gamma * jax.lax.rsqrt(var + _EPS)
    t = conv_b * s + beta - mean * s
    wt = jnp.transpose(w * s[:, None, None, None, None],
                       (2, 3, 4, 1, 0))
    band = jnp.zeros((3, 3, 4, C, 2, C), jnp.float32)
    for wq in range(2):
        for kw in range(3):
            band = band.at[:, :, wq + kw, :, wq, :].set(wt[:, :, kw])
    band = band.reshape(9, 4 * C, 2 * C).astype(jnp.bfloat16)
    tcol = jnp.concatenate([t, t]).reshape(1, 2 * C).astype(jnp.float32)
    return band, tcol


def kernel(x, w1, b1, gamma1, beta1, mean1, var1,
           w2, b2, gamma2, beta2, mean2, var2):
    xn = jnp.transpose(x, (0, 2, 3, 4, 1)).astype(jnp.float32)
    N, D, H, W, C = xn.shape
    band1, t1c = _fold_band(w1, b1, gamma1, beta1, mean1, var1, C)
    band2, t2c = _fold_band(w2, b2, gamma2, beta2, mean2, var2, C)
    NB = 8 if N % 8 == 0 else (4 if N % 4 == 0 else 2)
    out = _build_call(N, D, H, W, C, NB)(xn, band1, t1c, band2, t2c)
    return jnp.transpose(out, (0, 4, 1, 2, 3))
